# Initial kernel scaffold; baseline (speedup 1.0000x reference)
import jax
import jax.numpy as jnp
from jax.experimental import pallas as pl


def kernel(entity_ids, token_embeds, edge_index, edge_type, node_embeds, basis, comp, root, rgcn_bias, ep1_w1, ep1_b1, ep1_w2, ep1_b2, ep2_w, ep2_b, tp1_w1, tp1_b1, tp1_w2, tp1_b2, tp2_w, tp2_b, ca_w, pp1_w1, pp1_b1, pp1_w2, pp1_b2, pp2_w, pp2_b):
    raise NotImplementedError("write your pallas kernel here")


# speedup vs baseline: 1.1355x; 1.1355x over previous
import functools

import jax
import jax.numpy as jnp
from jax import lax
from jax.experimental import pallas as pl
from jax.experimental.pallas import tpu as pltpu

N_ENTITY = 10000
N_EDGES = 320000
NUM_REL = 24
NUM_BASES = 8
EHID = 128
HID = 256
THID = 768
N_LAYER = 12
N_BLOCK = 2
N_HEAD = 8
HEAD_DIM = HID // N_HEAD
B = 16
LE = 32
LT = 200
RKEY = 32


def _xw_body(comp_ref, basis_ref, x_ref, out_ref):
    r = pl.program_id(0)
    w_r = jnp.einsum("b,bio->io", comp_ref[r], basis_ref[...],
                     preferred_element_type=jnp.float32)
    out_ref[0] = jnp.dot(x_ref[...], w_r, preferred_element_type=jnp.float32)


def _compute_xw(comp, basis, x):
    return pl.pallas_call(
        _xw_body,
        grid=(NUM_REL,),
        in_specs=[
            pl.BlockSpec((NUM_REL, NUM_BASES), lambda r: (0, 0)),
            pl.BlockSpec((NUM_BASES, EHID, EHID), lambda r: (0, 0, 0)),
            pl.BlockSpec((N_ENTITY, EHID), lambda r: (0, 0)),
        ],
        out_specs=pl.BlockSpec((1, N_ENTITY, EHID), lambda r: (r, 0, 0)),
        out_shape=jax.ShapeDtypeStruct((NUM_REL, N_ENTITY, EHID), jnp.float32),
    )(comp, basis, x)


def _eidx_body(src_ref, dst_ref, rel_ref, key_ref, gidx_ref):
    rel = rel_ref[...]
    key_ref[...] = dst_ref[...] * RKEY + rel
    gidx_ref[...] = rel * N_ENTITY + src_ref[...]


def _edge_prep(edge_index, edge_type):
    rows = N_EDGES // 128
    src = edge_index[0].reshape(rows, 128)
    dst = edge_index[1].reshape(rows, 128)
    rel = edge_type.reshape(rows, 128)
    key, gidx = pl.pallas_call(
        _eidx_body,
        out_shape=[jax.ShapeDtypeStruct((rows, 128), jnp.int32)] * 2,
    )(src, dst, rel)
    return key.reshape(N_EDGES), gidx.reshape(N_EDGES)


def _edge_phase(xw, key, gidx, dst):
    cnt = jax.ops.segment_sum(jnp.ones((N_EDGES,), jnp.float32), key,
                              num_segments=N_ENTITY * RKEY)
    coef = 1.0 / jnp.maximum(cnt, 1.0)
    msg = xw.reshape(NUM_REL * N_ENTITY, EHID)[gidx] * coef[key][:, None]
    return jax.ops.segment_sum(msg, dst, num_segments=N_ENTITY)


def _ent_body(agg_ref, x_ref, root_ref, bias_ref, w1_ref, b1_ref, w2_ref,
              b2_ref, pw_ref, pb_ref, out_ref):
    x = x_ref[...]
    ent = (agg_ref[...] + jnp.dot(x, root_ref[...],
                                  preferred_element_type=jnp.float32)
           + bias_ref[...] + x)
    h = jax.nn.relu(jnp.dot(ent, w1_ref[...],
                            preferred_element_type=jnp.float32) + b1_ref[...])
    ent = jnp.dot(h, w2_ref[...], preferred_element_type=jnp.float32) \
        + b2_ref[...] + ent
    out_ref[...] = jnp.dot(ent, pw_ref[...],
                           preferred_element_type=jnp.float32) + pb_ref[...]


def _ent_path(agg, x, root, bias, w1, b1, w2, b2, pw, pb):
    blk = 2000
    full = lambda *s: pl.BlockSpec(s, lambda i: (0,) * len(s))
    return pl.pallas_call(
        _ent_body,
        grid=(N_ENTITY // blk,),
        in_specs=[
            pl.BlockSpec((blk, EHID), lambda i: (i, 0)),
            pl.BlockSpec((blk, EHID), lambda i: (i, 0)),
            full(EHID, EHID), full(EHID),
            full(EHID, EHID // 2), full(EHID // 2),
            full(EHID // 2, EHID), full(EHID),
            full(EHID, HID), full(HID),
        ],
        out_specs=pl.BlockSpec((blk, HID), lambda i: (i, 0)),
        out_shape=jax.ShapeDtypeStruct((N_ENTITY, HID), jnp.float32),
    )(agg, x, root, bias, w1, b1, w2, b2, pw, pb)


def _tok_body(t_ref, w1_ref, b1_ref, w2_ref, b2_ref, pw_ref, pb_ref, out_ref):
    t = t_ref[...]
    h = jax.nn.relu(jnp.dot(t, w1_ref[...],
                            preferred_element_type=jnp.float32) + b1_ref[...])
    tok = jnp.dot(h, w2_ref[...], preferred_element_type=jnp.float32) \
        + b2_ref[...] + t
    out_ref[...] = jnp.dot(tok, pw_ref[...],
                           preferred_element_type=jnp.float32) + pb_ref[...]


def _tok_path(tflat, w1, b1, w2, b2, pw, pb):
    blk = 800
    full = lambda *s: pl.BlockSpec(s, lambda i: (0,) * len(s))
    return pl.pallas_call(
        _tok_body,
        grid=(B * LT // blk,),
        in_specs=[
            pl.BlockSpec((blk, THID), lambda i: (i, 0)),
            full(THID, THID // 2), full(THID // 2),
            full(THID // 2, THID), full(THID),
            full(THID, HID), full(HID),
        ],
        out_specs=pl.BlockSpec((blk, HID), lambda i: (i, 0)),
        out_shape=jax.ShapeDtypeStruct((B * LT, HID), jnp.float32),
    )(tflat, w1, b1, w2, b2, pw, pb)


def _attn_body(tok_ref, es_ref, caw_ref, w1_ref, b1_ref, w2_ref, b2_ref,
               pw_ref, pb_ref, out_ref):
    tok = tok_ref[0]
    es = es_ref[0]
    q = jnp.dot(tok, caw_ref[...], preferred_element_type=jnp.float32)
    attn = jnp.dot(q, es.T, preferred_element_type=jnp.float32) / HID
    attn = attn - jnp.max(attn, axis=1, keepdims=True)
    e = jnp.exp(attn)
    ew = e / jnp.sum(e, axis=1, keepdims=True)
    prompt = jnp.dot(ew, es, preferred_element_type=jnp.float32) + tok
    h = jax.nn.relu(jnp.dot(prompt, w1_ref[...],
                            preferred_element_type=jnp.float32) + b1_ref[...])
    prompt = jnp.dot(h, w2_ref[...], preferred_element_type=jnp.float32) \
        + b2_ref[...] + prompt
    out_ref[0] = jnp.dot(prompt, pw_ref[...],
                         preferred_element_type=jnp.float32) + pb_ref[...]


def _attn_path(tok, ent_sel, ca_w, w1, b1, w2, b2, pw, pb):
    full = lambda *s: pl.BlockSpec(s, lambda i: (0,) * len(s))
    odim = N_LAYER * N_BLOCK * HID
    return pl.pallas_call(
        _attn_body,
        grid=(B,),
        in_specs=[
            pl.BlockSpec((1, LT, HID), lambda i: (i, 0, 0)),
            pl.BlockSpec((1, LE, HID), lambda i: (i, 0, 0)),
            full(HID, HID),
            full(HID, HID // 2), full(HID // 2),
            full(HID // 2, HID), full(HID),
            full(HID, odim), full(odim),
        ],
        out_specs=pl.BlockSpec((1, LT, odim), lambda i: (i, 0, 0)),
        out_shape=jax.ShapeDtypeStruct((B, LT, odim), jnp.float32),
    )(tok, ent_sel, ca_w, w1, b1, w2, b2, pw, pb)


def kernel(entity_ids, token_embeds, edge_index, edge_type, node_embeds,
           basis, comp, root, rgcn_bias,
           ep1_w1, ep1_b1, ep1_w2, ep1_b2, ep2_w, ep2_b,
           tp1_w1, tp1_b1, tp1_w2, tp1_b2, tp2_w, tp2_b,
           ca_w, pp1_w1, pp1_b1, pp1_w2, pp1_b2, pp2_w, pp2_b):
    xw = _compute_xw(comp, basis, node_embeds)
    key, gidx = _edge_prep(edge_index, edge_type)
    agg = _edge_phase(xw, key, gidx, edge_index[1])
    ent = _ent_path(agg, node_embeds, root, rgcn_bias,
                    ep1_w1, ep1_b1, ep1_w2, ep1_b2, ep2_w, ep2_b)
    ent_sel = ent[entity_ids]
    tok = _tok_path(token_embeds.reshape(B * LT, THID),
                    tp1_w1, tp1_b1, tp1_w2, tp1_b2, tp2_w, tp2_b)
    prompt = _attn_path(tok.reshape(B, LT, HID), ent_sel, ca_w,
                        pp1_w1, pp1_b1, pp1_w2, pp1_b2, pp2_w, pp2_b)
    prompt = prompt.reshape(B, LT, N_LAYER, N_BLOCK, N_HEAD, HEAD_DIM)
    return jnp.transpose(prompt, (2, 3, 0, 4, 1, 5))


# speedup vs baseline: 2.8181x; 2.4817x over previous
import functools

import jax
import jax.numpy as jnp
from jax import lax
from jax.experimental import pallas as pl
from jax.experimental.pallas import tpu as pltpu
from jax.experimental.pallas import tpu_sc as plsc

N_ENTITY = 10000
N_EDGES = 320000
NUM_REL = 24
NUM_BASES = 8
EHID = 128
HID = 256
THID = 768
N_LAYER = 12
N_BLOCK = 2
N_HEAD = 8
HEAD_DIM = HID // N_HEAD
B = 16
LE = 32
LT = 200
RKEY = 32


def _xw_body(comp_ref, basis_ref, x_ref, out_ref):
    r = pl.program_id(0)
    w_r = jnp.einsum("b,bio->io", comp_ref[r], basis_ref[...],
                     preferred_element_type=jnp.float32)
    out_ref[...] = jnp.dot(x_ref[...], w_r,
                           preferred_element_type=jnp.float32)


def _compute_xw(comp, basis, x):
    return pl.pallas_call(
        _xw_body,
        grid=(NUM_REL,),
        in_specs=[
            pl.BlockSpec((NUM_REL, NUM_BASES), lambda r: (0, 0)),
            pl.BlockSpec((NUM_BASES, EHID, EHID), lambda r: (0, 0, 0)),
            pl.BlockSpec((N_ENTITY, EHID), lambda r: (0, 0)),
        ],
        out_specs=pl.BlockSpec((N_ENTITY, EHID), lambda r: (r, 0)),
        out_shape=jax.ShapeDtypeStruct((NUM_REL * N_ENTITY, EHID),
                                       jnp.float32),
    )(comp, basis, x)


NW = 32
NDST = 320
SEGW = NDST * RKEY
NPAD = NW * NDST
CH1 = 4000
EPW = N_EDGES // NW
MCH = 80
ICH = 2000


def _e1_body(dst_hbm, rel_hbm, inv_hbm, dstb, relb, cnt, invloc):
    cid = lax.axis_index("c")
    sid = lax.axis_index("s")
    w = sid * 2 + cid
    base = w * SEGW
    zero16 = jnp.zeros((16,), jnp.float32)
    ones16 = jnp.ones((16,), jnp.float32)

    def z(i, _):
        cnt[pl.ds(i * 16, 16)] = zero16
        return 0
    lax.fori_loop(0, SEGW // 16, z, 0)

    def chunk(ci, _):
        pltpu.sync_copy(dst_hbm.at[pl.ds(ci * CH1, CH1)], dstb)
        pltpu.sync_copy(rel_hbm.at[pl.ds(ci * CH1, CH1)], relb)

        def it(i, _):
            loc = dstb[pl.ds(i * 16, 16)] * RKEY + relb[pl.ds(i * 16, 16)] \
                - base
            m = (loc >= 0) & (loc < SEGW)
            locc = jnp.minimum(jnp.maximum(loc, 0), SEGW - 1)
            plsc.addupdate_scatter(cnt, [locc], ones16, mask=m)
            return 0
        lax.fori_loop(0, CH1 // 16, it, 0)
        return 0
    lax.fori_loop(0, N_EDGES // CH1, chunk, 0)

    def inv_r(r, _):
        invloc[r, pl.ds(0, 16)] = 1.0 / jnp.maximum(cnt[pl.ds(r * RKEY, 16)],
                                                    1.0)
        invloc[r, pl.ds(16, 16)] = \
            1.0 / jnp.maximum(cnt[pl.ds(r * RKEY + 16, 16)], 1.0)
        return 0
    lax.fori_loop(0, NDST, inv_r, 0)
    pltpu.sync_copy(invloc, inv_hbm.at[pl.ds(w * NDST, NDST)])


def _e2_body(src_hbm, dst_hbm, rel_hbm, xw_hbm, inv_hbm, out_hbm,
             srcb, dstbig, relb, gidxb, dstb, msg, invb, agg_sh,
             sem1, sem2):
    cid = lax.axis_index("c")
    sid = lax.axis_index("s")
    w = sid * 2 + cid
    rows_pw = NPAD // 16
    iota16 = jnp.arange(16, dtype=jnp.int32)
    zero16 = jnp.zeros((16,), jnp.float32)

    def zrow(i, _):
        for j in range(8):
            msg[i, pl.ds(j * 16, 16)] = zero16
        return 0
    lax.fori_loop(0, MCH, zrow, 0)

    def zcp(i, _):
        pltpu.sync_copy(msg, agg_sh.at[pl.ds(sid * rows_pw + i * MCH, MCH)])
        return 0
    lax.fori_loop(0, rows_pw // MCH, zcp, 0)
    plsc.subcore_barrier()

    ebase = w * EPW

    def ichunk(ii, _):
        off = ebase + ii * ICH
        pltpu.sync_copy(src_hbm.at[pl.ds(off, ICH)], srcb)
        pltpu.sync_copy(dst_hbm.at[pl.ds(off, ICH)], dstbig)
        pltpu.sync_copy(rel_hbm.at[pl.ds(off, ICH)], relb)

        def mchunk(k, _):
            kb = k * MCH

            def bld(g, _):
                s16 = srcb[pl.ds(kb + g * 16, 16)]
                r16 = relb[pl.ds(kb + g * 16, 16)]
                gidxb[pl.ds(g * 16, 16)] = r16 * N_ENTITY + s16
                dstb[pl.ds(g * 16, 16)] = dstbig[pl.ds(kb + g * 16, 16)]
                return 0
            lax.fori_loop(0, MCH // 16, bld, 0)
            cp1 = pltpu.async_copy(xw_hbm.at[gidxb], msg, sem1)
            cp2 = pltpu.async_copy(inv_hbm.at[dstb], invb, sem2)
            cp1.wait()
            cp2.wait()

            def grp(g, _):
                r16 = relb[pl.ds(kb + g * 16, 16)]
                coef = plsc.load_gather(invb, [g * 16 + iota16, r16])
                for l in range(16):
                    c = coef[l]
                    j = g * 16 + l
                    for t in range(8):
                        msg[j, pl.ds(t * 16, 16)] = \
                            msg[j, pl.ds(t * 16, 16)] * c
                return 0
            lax.fori_loop(0, MCH // 16, grp, 0)
            pltpu.sync_copy(msg, agg_sh.at[dstb], add=True)
            return 0
        lax.fori_loop(0, ICH // MCH, mchunk, 0)
        return 0
    lax.fori_loop(0, EPW // ICH, ichunk, 0)
    plsc.subcore_barrier()
    pltpu.sync_copy(agg_sh.at[pl.ds(sid * rows_pw, rows_pw)],
                    out_hbm.at[cid, pl.ds(sid * rows_pw, rows_pw)])


def _edge_phase_sc(edge_index, edge_type, xw_flat):
    mesh = plsc.VectorSubcoreMesh(core_axis_name="c", subcore_axis_name="s")
    src = edge_index[0]
    dst = edge_index[1]
    invtab = pl.kernel(
        _e1_body,
        out_type=jax.ShapeDtypeStruct((NPAD, RKEY), jnp.float32),
        mesh=mesh,
        compiler_params=pltpu.CompilerParams(needs_layout_passes=False, use_tc_tiling_on_sc=False),
        scratch_types=[
            pltpu.VMEM((CH1,), jnp.int32),
            pltpu.VMEM((CH1,), jnp.int32),
            pltpu.VMEM((SEGW,), jnp.float32),
            pltpu.VMEM((NDST, RKEY), jnp.float32),
        ],
    )(dst, edge_type)
    agg2 = pl.kernel(
        _e2_body,
        out_type=jax.ShapeDtypeStruct((2, NPAD, EHID), jnp.float32),
        mesh=mesh,
        compiler_params=pltpu.CompilerParams(needs_layout_passes=False, use_tc_tiling_on_sc=False),
        scratch_types=[
            pltpu.VMEM((ICH,), jnp.int32),
            pltpu.VMEM((ICH,), jnp.int32),
            pltpu.VMEM((ICH,), jnp.int32),
            pltpu.VMEM((MCH,), jnp.int32),
            pltpu.VMEM((MCH,), jnp.int32),
            pltpu.VMEM((MCH, EHID), jnp.float32),
            pltpu.VMEM((MCH, RKEY), jnp.float32),
            pltpu.VMEM_SHARED((NPAD, EHID), jnp.float32),
            pltpu.SemaphoreType.DMA,
            pltpu.SemaphoreType.DMA,
        ],
    )(src, dst, edge_type, xw_flat, invtab)
    return agg2


def _ent_body(a0_ref, a1_ref, x_ref, root_ref, bias_ref, w1_ref, b1_ref,
              w2_ref, b2_ref, pw_ref, pb_ref, out_ref):
    x = x_ref[...]
    ent = (a0_ref[...] + a1_ref[...]
           + jnp.dot(x, root_ref[...], preferred_element_type=jnp.float32)
           + bias_ref[...] + x)
    h = jax.nn.relu(jnp.dot(ent, w1_ref[...],
                            preferred_element_type=jnp.float32) + b1_ref[...])
    ent = jnp.dot(h, w2_ref[...], preferred_element_type=jnp.float32) \
        + b2_ref[...] + ent
    out_ref[...] = jnp.dot(ent, pw_ref[...],
                           preferred_element_type=jnp.float32) + pb_ref[...]


def _ent_path(a0, a1, x, root, bias, w1, b1, w2, b2, pw, pb):
    blk = 2000
    full = lambda *s: pl.BlockSpec(s, lambda i: (0,) * len(s))
    return pl.pallas_call(
        _ent_body,
        grid=(N_ENTITY // blk,),
        in_specs=[
            pl.BlockSpec((blk, EHID), lambda i: (i, 0)),
            pl.BlockSpec((blk, EHID), lambda i: (i, 0)),
            pl.BlockSpec((blk, EHID), lambda i: (i, 0)),
            full(EHID, EHID), full(EHID),
            full(EHID, EHID // 2), full(EHID // 2),
            full(EHID // 2, EHID), full(EHID),
            full(EHID, HID), full(HID),
        ],
        out_specs=pl.BlockSpec((blk, HID), lambda i: (i, 0)),
        out_shape=jax.ShapeDtypeStruct((N_ENTITY, HID), jnp.float32),
    )(a0, a1, x, root, bias, w1, b1, w2, b2, pw, pb)


def _tok_body(t_ref, w1_ref, b1_ref, w2_ref, b2_ref, pw_ref, pb_ref, out_ref):
    t = t_ref[...]
    h = jax.nn.relu(jnp.dot(t, w1_ref[...],
                            preferred_element_type=jnp.float32) + b1_ref[...])
    tok = jnp.dot(h, w2_ref[...], preferred_element_type=jnp.float32) \
        + b2_ref[...] + t
    out_ref[...] = jnp.dot(tok, pw_ref[...],
                           preferred_element_type=jnp.float32) + pb_ref[...]


def _tok_path(tflat, w1, b1, w2, b2, pw, pb):
    blk = 800
    full = lambda *s: pl.BlockSpec(s, lambda i: (0,) * len(s))
    return pl.pallas_call(
        _tok_body,
        grid=(B * LT // blk,),
        in_specs=[
            pl.BlockSpec((blk, THID), lambda i: (i, 0)),
            full(THID, THID // 2), full(THID // 2),
            full(THID // 2, THID), full(THID),
            full(THID, HID), full(HID),
        ],
        out_specs=pl.BlockSpec((blk, HID), lambda i: (i, 0)),
        out_shape=jax.ShapeDtypeStruct((B * LT, HID), jnp.float32),
    )(tflat, w1, b1, w2, b2, pw, pb)


def _attn_body(tok_ref, es_ref, caw_ref, w1_ref, b1_ref, w2_ref, b2_ref,
               pw_ref, pb_ref, out_ref):
    tok = tok_ref[0]
    es = es_ref[0]
    q = jnp.dot(tok, caw_ref[...], preferred_element_type=jnp.float32)
    attn = jnp.dot(q, es.T, preferred_element_type=jnp.float32) / HID
    attn = attn - jnp.max(attn, axis=1, keepdims=True)
    e = jnp.exp(attn)
    ew = e / jnp.sum(e, axis=1, keepdims=True)
    prompt = jnp.dot(ew, es, preferred_element_type=jnp.float32) + tok
    h = jax.nn.relu(jnp.dot(prompt, w1_ref[...],
                            preferred_element_type=jnp.float32) + b1_ref[...])
    prompt = jnp.dot(h, w2_ref[...], preferred_element_type=jnp.float32) \
        + b2_ref[...] + prompt
    out_ref[0] = jnp.dot(prompt, pw_ref[...],
                         preferred_element_type=jnp.float32) + pb_ref[...]


def _attn_path(tok, ent_sel, ca_w, w1, b1, w2, b2, pw, pb):
    full = lambda *s: pl.BlockSpec(s, lambda i: (0,) * len(s))
    odim = N_LAYER * N_BLOCK * HID
    return pl.pallas_call(
        _attn_body,
        grid=(B,),
        in_specs=[
            pl.BlockSpec((1, LT, HID), lambda i: (i, 0, 0)),
            pl.BlockSpec((1, LE, HID), lambda i: (i, 0, 0)),
            full(HID, HID),
            full(HID, HID // 2), full(HID // 2),
            full(HID // 2, HID), full(HID),
            full(HID, odim), full(odim),
        ],
        out_specs=pl.BlockSpec((1, LT, odim), lambda i: (i, 0, 0)),
        out_shape=jax.ShapeDtypeStruct((B, LT, odim), jnp.float32),
    )(tok, ent_sel, ca_w, w1, b1, w2, b2, pw, pb)


def kernel(entity_ids, token_embeds, edge_index, edge_type, node_embeds,
           basis, comp, root, rgcn_bias,
           ep1_w1, ep1_b1, ep1_w2, ep1_b2, ep2_w, ep2_b,
           tp1_w1, tp1_b1, tp1_w2, tp1_b2, tp2_w, tp2_b,
           ca_w, pp1_w1, pp1_b1, pp1_w2, pp1_b2, pp2_w, pp2_b):
    xw = _compute_xw(comp, basis, node_embeds)
    agg2 = _edge_phase_sc(edge_index, edge_type, xw)
    ent = _ent_path(agg2[0], agg2[1], node_embeds, root, rgcn_bias,
                    ep1_w1, ep1_b1, ep1_w2, ep1_b2, ep2_w, ep2_b)
    ent_sel = ent[entity_ids]
    tok = _tok_path(token_embeds.reshape(B * LT, THID),
                    tp1_w1, tp1_b1, tp1_w2, tp1_b2, tp2_w, tp2_b)
    prompt = _attn_path(tok.reshape(B, LT, HID), ent_sel, ca_w,
                        pp1_w1, pp1_b1, pp1_w2, pp1_b2, pp2_w, pp2_b)
    prompt = prompt.reshape(B, LT, N_LAYER, N_BLOCK, N_HEAD, HEAD_DIM)
    return jnp.transpose(prompt, (2, 3, 0, 4, 1, 5))


# speedup vs baseline: 3.8541x; 1.3676x over previous
import functools

import jax
import jax.numpy as jnp
from jax import lax
from jax.experimental import pallas as pl
from jax.experimental.pallas import tpu as pltpu
from jax.experimental.pallas import tpu_sc as plsc

N_ENTITY = 10000
N_EDGES = 320000
NUM_REL = 24
NUM_BASES = 8
EHID = 128
HID = 256
THID = 768
N_LAYER = 12
N_BLOCK = 2
N_HEAD = 8
HEAD_DIM = HID // N_HEAD
B = 16
LE = 32
LT = 200
RKEY = 32


def _xw_body(comp_ref, basis_ref, x_ref, out_ref):
    r = pl.program_id(0)
    w_r = jnp.einsum("b,bio->io", comp_ref[r], basis_ref[...],
                     preferred_element_type=jnp.float32)
    out_ref[...] = jnp.dot(x_ref[...], w_r,
                           preferred_element_type=jnp.float32)


def _compute_xw(comp, basis, x):
    return pl.pallas_call(
        _xw_body,
        grid=(NUM_REL,),
        in_specs=[
            pl.BlockSpec((NUM_REL, NUM_BASES), lambda r: (0, 0)),
            pl.BlockSpec((NUM_BASES, EHID, EHID), lambda r: (0, 0, 0)),
            pl.BlockSpec((N_ENTITY, EHID), lambda r: (0, 0)),
        ],
        out_specs=pl.BlockSpec((N_ENTITY, EHID), lambda r: (r, 0)),
        out_shape=jax.ShapeDtypeStruct((NUM_REL * N_ENTITY, EHID),
                                       jnp.float32),
    )(comp, basis, x)


NW = 32
NDST = 320
SEGW = NDST * RKEY
NPAD = NW * NDST
CH1 = 4000
EPW = N_EDGES // NW
MCH = 80
ICH = 2000


def _eidx_body(src_ref, dst_ref, rel_ref, key_ref, gidx_ref):
    rel = rel_ref[...]
    key_ref[...] = dst_ref[...] * RKEY + rel
    gidx_ref[...] = rel * N_ENTITY + src_ref[...]


def _edge_prep(edge_index, edge_type):
    rows = N_EDGES // 128
    key, gidx = pl.pallas_call(
        _eidx_body,
        out_shape=[jax.ShapeDtypeStruct((rows, 128), jnp.int32)] * 2,
    )(edge_index[0].reshape(rows, 128), edge_index[1].reshape(rows, 128),
      edge_type.reshape(rows, 128))
    return key.reshape(N_EDGES), gidx.reshape(N_EDGES)


def _e1_body(key_hbm, inv_hbm, kb0, kb1, cnt, invloc, s0, s1):
    cid = lax.axis_index("c")
    sid = lax.axis_index("s")
    w = sid * 2 + cid
    base = w * SEGW
    zero16 = jnp.zeros((16,), jnp.float32)
    ones16 = jnp.ones((16,), jnp.float32)
    useg = jnp.uint32(SEGW)
    useg1 = jnp.uint32(SEGW - 1)

    def z(i, _):
        cnt[pl.ds(i * 16, 16)] = zero16
        return 0
    lax.fori_loop(0, SEGW // 16, z, 0)

    nch = N_EDGES // CH1
    kbufs = (kb0, kb1)
    sems = (s0, s1)
    cps = {
        0: pltpu.async_copy(key_hbm.at[pl.ds(0, CH1)], kb0, s0),
        1: pltpu.async_copy(key_hbm.at[pl.ds(CH1, CH1)], kb1, s1),
    }
    for ci in range(nch):
        b = ci % 2
        kb = kbufs[b]
        cps.pop(ci).wait()

        def it(i, _):
            lu = (kb[pl.ds(i * 16, 16)] - base).astype(jnp.uint32)
            m = lu < useg
            locc = jnp.minimum(lu, useg1).astype(jnp.int32)
            plsc.addupdate_scatter(cnt, [locc], ones16, mask=m)
            return 0
        lax.fori_loop(0, CH1 // 16, it, 0)
        if ci + 2 < nch:
            cps[ci + 2] = pltpu.async_copy(
                key_hbm.at[pl.ds((ci + 2) * CH1, CH1)], kb, sems[b])

    def inv_r(r, _):
        invloc[r, pl.ds(0, 16)] = 1.0 / jnp.maximum(cnt[pl.ds(r * RKEY, 16)],
                                                    1.0)
        invloc[r, pl.ds(16, 16)] = \
            1.0 / jnp.maximum(cnt[pl.ds(r * RKEY + 16, 16)], 1.0)
        return 0
    lax.fori_loop(0, NDST, inv_r, 0)
    pltpu.sync_copy(invloc, inv_hbm.at[pl.ds(w * NDST, NDST)])


def _e2_body(gidx_hbm, dst_hbm, rel_hbm, xw_hbm, inv_hbm, out_hbm,
             gb0, gb1, db0, db1, rb0, rb1, m0, m1, iv0, iv1, agg_sh,
             sg0, sg1, si0, si1, sx0, sx1):
    cid = lax.axis_index("c")
    sid = lax.axis_index("s")
    w = sid * 2 + cid
    rows_pw = NPAD // 16
    iota16 = jnp.arange(16, dtype=jnp.int32)
    zero16 = jnp.zeros((16,), jnp.float32)
    nch = EPW // MCH
    ebase = w * EPW
    gbufs, dbufs, rbufs = (gb0, gb1), (db0, db1), (rb0, rb1)
    msgs, invs = (m0, m1), (iv0, iv1)
    sgs, sis, sxs = (sg0, sg1), (si0, si1), (sx0, sx1)

    def issue_idx(k, b):
        off = ebase + k * MCH
        pltpu.async_copy(gidx_hbm.at[pl.ds(off, MCH)], gbufs[b], sxs[b])
        pltpu.async_copy(dst_hbm.at[pl.ds(off, MCH)], dbufs[b], sxs[b])
        pltpu.async_copy(rel_hbm.at[pl.ds(off, MCH)], rbufs[b], sxs[b])

    def wait_idx(b):
        pltpu.make_async_copy(gidx_hbm.at[pl.ds(0, MCH)], gbufs[b],
                              sxs[b]).wait()
        pltpu.make_async_copy(dst_hbm.at[pl.ds(0, MCH)], dbufs[b],
                              sxs[b]).wait()
        pltpu.make_async_copy(rel_hbm.at[pl.ds(0, MCH)], rbufs[b],
                              sxs[b]).wait()

    def issue_gath(b):
        pltpu.async_copy(xw_hbm.at[gbufs[b]], msgs[b], sgs[b])
        pltpu.async_copy(inv_hbm.at[dbufs[b]], invs[b], sis[b])

    def wait_gath(b):
        pltpu.make_async_copy(xw_hbm.at[gbufs[b]], msgs[b], sgs[b]).wait()
        pltpu.make_async_copy(inv_hbm.at[dbufs[b]], invs[b], sis[b]).wait()

    def scale(b):
        msg = msgs[b]

        def grp(g, _):
            r16 = rbufs[b][pl.ds(g * 16, 16)]
            coef = plsc.load_gather(invs[b], [g * 16 + iota16, r16])
            for l in range(16):
                c = coef[l]
                j = g * 16 + l
                for t in range(8):
                    msg[j, pl.ds(t * 16, 16)] = msg[j, pl.ds(t * 16, 16)] * c
            return 0
        lax.fori_loop(0, MCH // 16, grp, 0)

    def chunk_body(k, b, has_next, has_next2):
        wait_gath(b)
        if has_next:
            wait_idx(1 - b)
            issue_gath(1 - b)
        scale(b)
        pltpu.sync_copy(msgs[b], agg_sh.at[dbufs[b]], add=True)
        if has_next2:
            issue_idx(k + 2, b)

    def zrow(i, _):
        for j in range(8):
            m0[i, pl.ds(j * 16, 16)] = zero16
        return 0
    lax.fori_loop(0, MCH, zrow, 0)
    issue_idx(0, 0)
    issue_idx(1, 1)

    def zcp(i, _):
        pltpu.sync_copy(m0, agg_sh.at[pl.ds(sid * rows_pw + i * MCH, MCH)])
        return 0
    lax.fori_loop(0, rows_pw // MCH, zcp, 0)
    plsc.subcore_barrier()

    wait_idx(0)
    issue_gath(0)

    def pair(i, _):
        chunk_body(2 * i, 0, True, True)
        chunk_body(2 * i + 1, 1, True, True)
        return 0
    lax.fori_loop(0, nch // 2 - 1, pair, 0)
    chunk_body(nch - 3, 0, True, True)
    chunk_body(nch - 2, 1, True, False)
    chunk_body(nch - 1, 0, False, False)

    plsc.subcore_barrier()
    pltpu.sync_copy(agg_sh.at[pl.ds(sid * rows_pw, rows_pw)],
                    out_hbm.at[cid, pl.ds(sid * rows_pw, rows_pw)])


def _edge_phase_sc(edge_index, edge_type, xw_flat):
    mesh = plsc.VectorSubcoreMesh(core_axis_name="c", subcore_axis_name="s")
    params = pltpu.CompilerParams(needs_layout_passes=False,
                                  use_tc_tiling_on_sc=False)
    key, gidx = _edge_prep(edge_index, edge_type)
    dst = edge_index[1]
    invtab = pl.kernel(
        _e1_body,
        out_type=jax.ShapeDtypeStruct((NPAD, RKEY), jnp.float32),
        mesh=mesh,
        compiler_params=params,
        scratch_types=[
            pltpu.VMEM((CH1,), jnp.int32),
            pltpu.VMEM((CH1,), jnp.int32),
            pltpu.VMEM((SEGW,), jnp.float32),
            pltpu.VMEM((NDST, RKEY), jnp.float32),
            pltpu.SemaphoreType.DMA,
            pltpu.SemaphoreType.DMA,
        ],
    )(key)
    agg2 = pl.kernel(
        _e2_body,
        out_type=jax.ShapeDtypeStruct((2, NPAD, EHID), jnp.float32),
        mesh=mesh,
        compiler_params=params,
        scratch_types=[
            pltpu.VMEM((MCH,), jnp.int32),
            pltpu.VMEM((MCH,), jnp.int32),
            pltpu.VMEM((MCH,), jnp.int32),
            pltpu.VMEM((MCH,), jnp.int32),
            pltpu.VMEM((MCH,), jnp.int32),
            pltpu.VMEM((MCH,), jnp.int32),
            pltpu.VMEM((MCH, EHID), jnp.float32),
            pltpu.VMEM((MCH, EHID), jnp.float32),
            pltpu.VMEM((MCH, RKEY), jnp.float32),
            pltpu.VMEM((MCH, RKEY), jnp.float32),
            pltpu.VMEM_SHARED((NPAD, EHID), jnp.float32),
            pltpu.SemaphoreType.DMA,
            pltpu.SemaphoreType.DMA,
            pltpu.SemaphoreType.DMA,
            pltpu.SemaphoreType.DMA,
            pltpu.SemaphoreType.DMA,
            pltpu.SemaphoreType.DMA,
        ],
    )(gidx, dst, edge_type, xw_flat, invtab)
    return agg2


def _ent_body(a0_ref, a1_ref, x_ref, root_ref, bias_ref, w1_ref, b1_ref,
              w2_ref, b2_ref, pw_ref, pb_ref, out_ref):
    x = x_ref[...]
    ent = (a0_ref[...] + a1_ref[...]
           + jnp.dot(x, root_ref[...], preferred_element_type=jnp.float32)
           + bias_ref[...] + x)
    h = jax.nn.relu(jnp.dot(ent, w1_ref[...],
                            preferred_element_type=jnp.float32) + b1_ref[...])
    ent = jnp.dot(h, w2_ref[...], preferred_element_type=jnp.float32) \
        + b2_ref[...] + ent
    out_ref[...] = jnp.dot(ent, pw_ref[...],
                           preferred_element_type=jnp.float32) + pb_ref[...]


def _ent_path(a0, a1, x, root, bias, w1, b1, w2, b2, pw, pb):
    blk = 2000
    full = lambda *s: pl.BlockSpec(s, lambda i: (0,) * len(s))
    return pl.pallas_call(
        _ent_body,
        grid=(N_ENTITY // blk,),
        in_specs=[
            pl.BlockSpec((blk, EHID), lambda i: (i, 0)),
            pl.BlockSpec((blk, EHID), lambda i: (i, 0)),
            pl.BlockSpec((blk, EHID), lambda i: (i, 0)),
            full(EHID, EHID), full(EHID),
            full(EHID, EHID // 2), full(EHID // 2),
            full(EHID // 2, EHID), full(EHID),
            full(EHID, HID), full(HID),
        ],
        out_specs=pl.BlockSpec((blk, HID), lambda i: (i, 0)),
        out_shape=jax.ShapeDtypeStruct((N_ENTITY, HID), jnp.float32),
    )(a0, a1, x, root, bias, w1, b1, w2, b2, pw, pb)


def _tok_body(t_ref, w1_ref, b1_ref, w2_ref, b2_ref, pw_ref, pb_ref, out_ref):
    t = t_ref[...]
    h = jax.nn.relu(jnp.dot(t, w1_ref[...],
                            preferred_element_type=jnp.float32) + b1_ref[...])
    tok = jnp.dot(h, w2_ref[...], preferred_element_type=jnp.float32) \
        + b2_ref[...] + t
    out_ref[...] = jnp.dot(tok, pw_ref[...],
                           preferred_element_type=jnp.float32) + pb_ref[...]


def _tok_path(tflat, w1, b1, w2, b2, pw, pb):
    blk = 800
    full = lambda *s: pl.BlockSpec(s, lambda i: (0,) * len(s))
    return pl.pallas_call(
        _tok_body,
        grid=(B * LT // blk,),
        in_specs=[
            pl.BlockSpec((blk, THID), lambda i: (i, 0)),
            full(THID, THID // 2), full(THID // 2),
            full(THID // 2, THID), full(THID),
            full(THID, HID), full(HID),
        ],
        out_specs=pl.BlockSpec((blk, HID), lambda i: (i, 0)),
        out_shape=jax.ShapeDtypeStruct((B * LT, HID), jnp.float32),
    )(tflat, w1, b1, w2, b2, pw, pb)


def _attn_body(tok_ref, es_ref, caw_ref, w1_ref, b1_ref, w2_ref, b2_ref,
               pw_ref, pb_ref, out_ref):
    tok = tok_ref[0]
    es = es_ref[0]
    q = jnp.dot(tok, caw_ref[...], preferred_element_type=jnp.float32)
    attn = jnp.dot(q, es.T, preferred_element_type=jnp.float32) / HID
    attn = attn - jnp.max(attn, axis=1, keepdims=True)
    e = jnp.exp(attn)
    ew = e / jnp.sum(e, axis=1, keepdims=True)
    prompt = jnp.dot(ew, es, preferred_element_type=jnp.float32) + tok
    h = jax.nn.relu(jnp.dot(prompt, w1_ref[...],
                            preferred_element_type=jnp.float32) + b1_ref[...])
    prompt = jnp.dot(h, w2_ref[...], preferred_element_type=jnp.float32) \
        + b2_ref[...] + prompt
    out_ref[0] = jnp.dot(prompt, pw_ref[...],
                         preferred_element_type=jnp.float32) + pb_ref[...]


def _attn_path(tok, ent_sel, ca_w, w1, b1, w2, b2, pw, pb):
    full = lambda *s: pl.BlockSpec(s, lambda i: (0,) * len(s))
    odim = N_LAYER * N_BLOCK * HID
    return pl.pallas_call(
        _attn_body,
        grid=(B,),
        in_specs=[
            pl.BlockSpec((1, LT, HID), lambda i: (i, 0, 0)),
            pl.BlockSpec((1, LE, HID), lambda i: (i, 0, 0)),
            full(HID, HID),
            full(HID, HID // 2), full(HID // 2),
            full(HID // 2, HID), full(HID),
            full(HID, odim), full(odim),
        ],
        out_specs=pl.BlockSpec((1, LT, odim), lambda i: (i, 0, 0)),
        out_shape=jax.ShapeDtypeStruct((B, LT, odim), jnp.float32),
    )(tok, ent_sel, ca_w, w1, b1, w2, b2, pw, pb)


def kernel(entity_ids, token_embeds, edge_index, edge_type, node_embeds,
           basis, comp, root, rgcn_bias,
           ep1_w1, ep1_b1, ep1_w2, ep1_b2, ep2_w, ep2_b,
           tp1_w1, tp1_b1, tp1_w2, tp1_b2, tp2_w, tp2_b,
           ca_w, pp1_w1, pp1_b1, pp1_w2, pp1_b2, pp2_w, pp2_b):
    xw = _compute_xw(comp, basis, node_embeds)
    agg2 = _edge_phase_sc(edge_index, edge_type, xw)
    ent = _ent_path(agg2[0], agg2[1], node_embeds, root, rgcn_bias,
                    ep1_w1, ep1_b1, ep1_w2, ep1_b2, ep2_w, ep2_b)
    ent_sel = ent[entity_ids]
    tok = _tok_path(token_embeds.reshape(B * LT, THID),
                    tp1_w1, tp1_b1, tp1_w2, tp1_b2, tp2_w, tp2_b)
    prompt = _attn_path(tok.reshape(B, LT, HID), ent_sel, ca_w,
                        pp1_w1, pp1_b1, pp1_w2, pp1_b2, pp2_w, pp2_b)
    prompt = prompt.reshape(B, LT, N_LAYER, N_BLOCK, N_HEAD, HEAD_DIM)
    return jnp.transpose(prompt, (2, 3, 0, 4, 1, 5))


# speedup vs baseline: 4.5427x; 1.1787x over previous
import functools

import jax
import jax.numpy as jnp
from jax import lax
from jax.experimental import pallas as pl
from jax.experimental.pallas import tpu as pltpu
from jax.experimental.pallas import tpu_sc as plsc

N_ENTITY = 10000
N_EDGES = 320000
NUM_REL = 24
NUM_BASES = 8
EHID = 128
HID = 256
THID = 768
N_LAYER = 12
N_BLOCK = 2
N_HEAD = 8
HEAD_DIM = HID // N_HEAD
B = 16
LE = 32
LT = 200
RKEY = 32


def _xw_body(comp_ref, basis_ref, x_ref, out_ref):
    r = pl.program_id(0)
    w_r = jnp.einsum("b,bio->io", comp_ref[r], basis_ref[...],
                     preferred_element_type=jnp.float32)
    out_ref[...] = jnp.dot(x_ref[...], w_r,
                           preferred_element_type=jnp.float32)


def _compute_xw(comp, basis, x):
    return pl.pallas_call(
        _xw_body,
        grid=(NUM_REL,),
        in_specs=[
            pl.BlockSpec((NUM_REL, NUM_BASES), lambda r: (0, 0)),
            pl.BlockSpec((NUM_BASES, EHID, EHID), lambda r: (0, 0, 0)),
            pl.BlockSpec((N_ENTITY, EHID), lambda r: (0, 0)),
        ],
        out_specs=pl.BlockSpec((N_ENTITY, EHID), lambda r: (r, 0)),
        out_shape=jax.ShapeDtypeStruct((NUM_REL * N_ENTITY, EHID),
                                       jnp.float32),
    )(comp, basis, x)


NW = 32
NDST = 320
SEGW = NDST * RKEY
NPAD = NW * NDST
CH1 = 4000
EPW = N_EDGES // NW
MCH = 80
ICH = 2000


def _eidx_body(src_ref, dst_ref, rel_ref, key_ref, gidx_ref):
    rel = rel_ref[...]
    key_ref[...] = dst_ref[...] * RKEY + rel
    gidx_ref[...] = rel * N_ENTITY + src_ref[...]


def _edge_prep(edge_index, edge_type):
    rows = N_EDGES // 128
    key, gidx = pl.pallas_call(
        _eidx_body,
        out_shape=[jax.ShapeDtypeStruct((rows, 128), jnp.int32)] * 2,
    )(edge_index[0].reshape(rows, 128), edge_index[1].reshape(rows, 128),
      edge_type.reshape(rows, 128))
    return key.reshape(N_EDGES), gidx.reshape(N_EDGES)


def _e1_body(key_hbm, inv_hbm, kb0, kb1, cnt, invloc, s0, s1):
    cid = lax.axis_index("c")
    sid = lax.axis_index("s")
    w = sid * 2 + cid
    base = w * SEGW
    zero16 = jnp.zeros((16,), jnp.float32)
    ones16 = jnp.ones((16,), jnp.float32)
    useg = jnp.uint32(SEGW)
    useg1 = jnp.uint32(SEGW - 1)

    def z(i, _):
        cnt[pl.ds(i * 16, 16)] = zero16
        return 0
    lax.fori_loop(0, SEGW // 16, z, 0)

    nch = N_EDGES // CH1
    kbufs = (kb0, kb1)
    sems = (s0, s1)
    cps = {
        0: pltpu.async_copy(key_hbm.at[pl.ds(0, CH1)], kb0, s0),
        1: pltpu.async_copy(key_hbm.at[pl.ds(CH1, CH1)], kb1, s1),
    }
    for ci in range(nch):
        b = ci % 2
        kb = kbufs[b]
        cps.pop(ci).wait()

        @plsc.parallel_loop(0, CH1 // 16, unroll=8)
        def _(i):
            lu = (kb[pl.ds(i * 16, 16)] - base).astype(jnp.uint32)
            m = lu < useg
            locc = jnp.minimum(lu, useg1).astype(jnp.int32)
            plsc.addupdate_scatter(cnt, [locc], ones16, mask=m)
        if ci + 2 < nch:
            cps[ci + 2] = pltpu.async_copy(
                key_hbm.at[pl.ds((ci + 2) * CH1, CH1)], kb, sems[b])

    def inv_r(r, _):
        invloc[r, pl.ds(0, 16)] = 1.0 / jnp.maximum(cnt[pl.ds(r * RKEY, 16)],
                                                    1.0)
        invloc[r, pl.ds(16, 16)] = \
            1.0 / jnp.maximum(cnt[pl.ds(r * RKEY + 16, 16)], 1.0)
        return 0
    lax.fori_loop(0, NDST, inv_r, 0)
    pltpu.sync_copy(invloc, inv_hbm.at[pl.ds(w * NDST, NDST)])


def _e2_body(gidx_hbm, dst_hbm, rel_hbm, xw_hbm, inv_hbm, out_hbm,
             gb0, gb1, gb2, db0, db1, db2, rb0, rb1, rb2,
             m0, m1, m2, iv0, iv1, iv2, sd0, sd1, agg_sh,
             sg0, sg1, sg2, si0, si1, si2, sx0, sx1, sx2, ss0, ss1):
    cid = lax.axis_index("c")
    sid = lax.axis_index("s")
    w = sid * 2 + cid
    rows_pw = NPAD // 16
    iota16 = jnp.arange(16, dtype=jnp.int32)
    zero16 = jnp.zeros((16,), jnp.float32)
    nch = EPW // MCH
    ebase = w * EPW
    gbufs, dbufs, rbufs = (gb0, gb1, gb2), (db0, db1, db2), (rb0, rb1, rb2)
    msgs, invs = (m0, m1, m2), (iv0, iv1, iv2)
    sdbs = (sd0, sd1)
    sgs, sis, sxs, sscs = (sg0, sg1, sg2), (si0, si1, si2), \
        (sx0, sx1, sx2), (ss0, ss1)

    def issue_idx(k, b):
        off = ebase + k * MCH
        pltpu.async_copy(gidx_hbm.at[pl.ds(off, MCH)], gbufs[b], sxs[b])
        pltpu.async_copy(dst_hbm.at[pl.ds(off, MCH)], dbufs[b], sxs[b])
        pltpu.async_copy(rel_hbm.at[pl.ds(off, MCH)], rbufs[b], sxs[b])

    def wait_idx(b):
        pltpu.make_async_copy(gidx_hbm.at[pl.ds(0, MCH)], gbufs[b],
                              sxs[b]).wait()
        pltpu.make_async_copy(dst_hbm.at[pl.ds(0, MCH)], dbufs[b],
                              sxs[b]).wait()
        pltpu.make_async_copy(rel_hbm.at[pl.ds(0, MCH)], rbufs[b],
                              sxs[b]).wait()

    def issue_gath(b):
        pltpu.async_copy(xw_hbm.at[gbufs[b]], msgs[b], sgs[b])
        pltpu.async_copy(inv_hbm.at[dbufs[b]], invs[b], sis[b])

    def wait_gath(b):
        pltpu.make_async_copy(xw_hbm.at[gbufs[b]], msgs[b], sgs[b]).wait()
        pltpu.make_async_copy(inv_hbm.at[dbufs[b]], invs[b], sis[b]).wait()

    def wait_scat(b2):
        pltpu.make_async_copy(msgs[0], agg_sh.at[sdbs[b2]], sscs[b2]).wait()

    def scale(b):
        msg = msgs[b]

        @plsc.parallel_loop(0, MCH // 16, unroll=1)
        def _(g):
            r16 = rbufs[b][pl.ds(g * 16, 16)]
            coef = plsc.load_gather(invs[b], [g * 16 + iota16, r16])
            for l in range(16):
                c = coef[l]
                j = g * 16 + l
                for t in range(8):
                    msg[j, pl.ds(t * 16, 16)] = msg[j, pl.ds(t * 16, 16)] * c

    def chunk_body(k, b, b2, first, last):
        wait_gath(b)
        if not first:
            wait_scat(b2)
        if not last:
            wait_idx((b + 1) % 3)
            issue_gath((b + 1) % 3)

        def cpd(g, _):
            sdbs[b2][pl.ds(g * 16, 16)] = dbufs[b][pl.ds(g * 16, 16)]
            return 0
        lax.fori_loop(0, MCH // 16, cpd, 0)
        scale(b)
        pltpu.async_copy(msgs[b], agg_sh.at[sdbs[b2]], sscs[b2], add=True)
        if k is not None:
            issue_idx(k + 2, (b + 2) % 3)

    def zrow(i, _):
        for j in range(8):
            m0[i, pl.ds(j * 16, 16)] = zero16
        return 0
    lax.fori_loop(0, MCH, zrow, 0)
    issue_idx(0, 0)
    issue_idx(1, 1)

    def zcp(i, _):
        pltpu.sync_copy(m0, agg_sh.at[pl.ds(sid * rows_pw + i * MCH, MCH)])
        return 0
    lax.fori_loop(0, rows_pw // MCH, zcp, 0)
    plsc.subcore_barrier()

    wait_idx(0)
    issue_gath(0)
    chunk_body(0, 0, 0, True, False)
    chunk_body(1, 1, 1, True, False)

    def six(i, _):
        k = 2 + 6 * i
        for u in range(6):
            chunk_body(k + u, (2 + u) % 3, u % 2, False, False)
        return 0
    lax.fori_loop(0, (nch - 5) // 6, six, 0)
    chunk_body(122, (122 % 3), 0, False, False)
    chunk_body(None, (123 % 3), 1, False, False)
    b124, p124 = 124 % 3, 0
    wait_gath(b124)
    wait_scat(p124)

    def cpd_t(g, _):
        sdbs[p124][pl.ds(g * 16, 16)] = dbufs[b124][pl.ds(g * 16, 16)]
        return 0
    lax.fori_loop(0, MCH // 16, cpd_t, 0)
    scale(b124)
    pltpu.async_copy(msgs[b124], agg_sh.at[sdbs[p124]], sscs[p124], add=True)
    wait_scat(1)
    wait_scat(0)

    plsc.subcore_barrier()
    pltpu.sync_copy(agg_sh.at[pl.ds(sid * rows_pw, rows_pw)],
                    out_hbm.at[cid, pl.ds(sid * rows_pw, rows_pw)])


def _edge_phase_sc(edge_index, edge_type, xw_flat):
    mesh = plsc.VectorSubcoreMesh(core_axis_name="c", subcore_axis_name="s")
    params = pltpu.CompilerParams(needs_layout_passes=False,
                                  use_tc_tiling_on_sc=False)
    key, gidx = _edge_prep(edge_index, edge_type)
    dst = edge_index[1]
    invtab = pl.kernel(
        _e1_body,
        out_type=jax.ShapeDtypeStruct((NPAD, RKEY), jnp.float32),
        mesh=mesh,
        compiler_params=params,
        scratch_types=[
            pltpu.VMEM((CH1,), jnp.int32),
            pltpu.VMEM((CH1,), jnp.int32),
            pltpu.VMEM((SEGW,), jnp.float32),
            pltpu.VMEM((NDST, RKEY), jnp.float32),
            pltpu.SemaphoreType.DMA,
            pltpu.SemaphoreType.DMA,
        ],
    )(key)
    agg2 = pl.kernel(
        _e2_body,
        out_type=jax.ShapeDtypeStruct((2, NPAD, EHID), jnp.float32),
        mesh=mesh,
        compiler_params=params,
        scratch_types=(
            [pltpu.VMEM((MCH,), jnp.int32)] * 9
            + [pltpu.VMEM((MCH, EHID), jnp.float32)] * 3
            + [pltpu.VMEM((MCH, RKEY), jnp.float32)] * 3
            + [pltpu.VMEM((MCH,), jnp.int32)] * 2
            + [pltpu.VMEM_SHARED((NPAD, EHID), jnp.float32)]
            + [pltpu.SemaphoreType.DMA] * 11
        ),
    )(gidx, dst, edge_type, xw_flat, invtab)
    return agg2


def _ent_body(a0_ref, a1_ref, x_ref, root_ref, bias_ref, w1_ref, b1_ref,
              w2_ref, b2_ref, pw_ref, pb_ref, out_ref):
    x = x_ref[...]
    ent = (a0_ref[...] + a1_ref[...]
           + jnp.dot(x, root_ref[...], preferred_element_type=jnp.float32)
           + bias_ref[...] + x)
    h = jax.nn.relu(jnp.dot(ent, w1_ref[...],
                            preferred_element_type=jnp.float32) + b1_ref[...])
    ent = jnp.dot(h, w2_ref[...], preferred_element_type=jnp.float32) \
        + b2_ref[...] + ent
    out_ref[...] = jnp.dot(ent, pw_ref[...],
                           preferred_element_type=jnp.float32) + pb_ref[...]


def _ent_path(a0, a1, x, root, bias, w1, b1, w2, b2, pw, pb):
    blk = 2000
    full = lambda *s: pl.BlockSpec(s, lambda i: (0,) * len(s))
    return pl.pallas_call(
        _ent_body,
        grid=(N_ENTITY // blk,),
        in_specs=[
            pl.BlockSpec((blk, EHID), lambda i: (i, 0)),
            pl.BlockSpec((blk, EHID), lambda i: (i, 0)),
            pl.BlockSpec((blk, EHID), lambda i: (i, 0)),
            full(EHID, EHID), full(EHID),
            full(EHID, EHID // 2), full(EHID // 2),
            full(EHID // 2, EHID), full(EHID),
            full(EHID, HID), full(HID),
        ],
        out_specs=pl.BlockSpec((blk, HID), lambda i: (i, 0)),
        out_shape=jax.ShapeDtypeStruct((N_ENTITY, HID), jnp.float32),
    )(a0, a1, x, root, bias, w1, b1, w2, b2, pw, pb)


def _tok_body(t_ref, w1_ref, b1_ref, w2_ref, b2_ref, pw_ref, pb_ref, out_ref):
    t = t_ref[...]
    h = jax.nn.relu(jnp.dot(t, w1_ref[...],
                            preferred_element_type=jnp.float32) + b1_ref[...])
    tok = jnp.dot(h, w2_ref[...], preferred_element_type=jnp.float32) \
        + b2_ref[...] + t
    out_ref[...] = jnp.dot(tok, pw_ref[...],
                           preferred_element_type=jnp.float32) + pb_ref[...]


def _tok_path(tflat, w1, b1, w2, b2, pw, pb):
    blk = 800
    full = lambda *s: pl.BlockSpec(s, lambda i: (0,) * len(s))
    return pl.pallas_call(
        _tok_body,
        grid=(B * LT // blk,),
        in_specs=[
            pl.BlockSpec((blk, THID), lambda i: (i, 0)),
            full(THID, THID // 2), full(THID // 2),
            full(THID // 2, THID), full(THID),
            full(THID, HID), full(HID),
        ],
        out_specs=pl.BlockSpec((blk, HID), lambda i: (i, 0)),
        out_shape=jax.ShapeDtypeStruct((B * LT, HID), jnp.float32),
    )(tflat, w1, b1, w2, b2, pw, pb)


def _attn_body(tok_ref, es_ref, caw_ref, w1_ref, b1_ref, w2_ref, b2_ref,
               pw_ref, pb_ref, out_ref):
    tok = tok_ref[0]
    es = es_ref[0]
    q = jnp.dot(tok, caw_ref[...], preferred_element_type=jnp.float32)
    attn = jnp.dot(q, es.T, preferred_element_type=jnp.float32) / HID
    attn = attn - jnp.max(attn, axis=1, keepdims=True)
    e = jnp.exp(attn)
    ew = e / jnp.sum(e, axis=1, keepdims=True)
    prompt = jnp.dot(ew, es, preferred_element_type=jnp.float32) + tok
    h = jax.nn.relu(jnp.dot(prompt, w1_ref[...],
                            preferred_element_type=jnp.float32) + b1_ref[...])
    prompt = jnp.dot(h, w2_ref[...], preferred_element_type=jnp.float32) \
        + b2_ref[...] + prompt
    out_ref[0] = jnp.dot(prompt, pw_ref[...],
                         preferred_element_type=jnp.float32) + pb_ref[...]


def _attn_path(tok, ent_sel, ca_w, w1, b1, w2, b2, pw, pb):
    full = lambda *s: pl.BlockSpec(s, lambda i: (0,) * len(s))
    odim = N_LAYER * N_BLOCK * HID
    return pl.pallas_call(
        _attn_body,
        grid=(B,),
        in_specs=[
            pl.BlockSpec((1, LT, HID), lambda i: (i, 0, 0)),
            pl.BlockSpec((1, LE, HID), lambda i: (i, 0, 0)),
            full(HID, HID),
            full(HID, HID // 2), full(HID // 2),
            full(HID // 2, HID), full(HID),
            full(HID, odim), full(odim),
        ],
        out_specs=pl.BlockSpec((1, LT, odim), lambda i: (i, 0, 0)),
        out_shape=jax.ShapeDtypeStruct((B, LT, odim), jnp.float32),
    )(tok, ent_sel, ca_w, w1, b1, w2, b2, pw, pb)


def kernel(entity_ids, token_embeds, edge_index, edge_type, node_embeds,
           basis, comp, root, rgcn_bias,
           ep1_w1, ep1_b1, ep1_w2, ep1_b2, ep2_w, ep2_b,
           tp1_w1, tp1_b1, tp1_w2, tp1_b2, tp2_w, tp2_b,
           ca_w, pp1_w1, pp1_b1, pp1_w2, pp1_b2, pp2_w, pp2_b):
    xw = _compute_xw(comp, basis, node_embeds)
    agg2 = _edge_phase_sc(edge_index, edge_type, xw)
    ent = _ent_path(agg2[0], agg2[1], node_embeds, root, rgcn_bias,
                    ep1_w1, ep1_b1, ep1_w2, ep1_b2, ep2_w, ep2_b)
    ent_sel = ent[entity_ids]
    tok = _tok_path(token_embeds.reshape(B * LT, THID),
                    tp1_w1, tp1_b1, tp1_w2, tp1_b2, tp2_w, tp2_b)
    prompt = _attn_path(tok.reshape(B, LT, HID), ent_sel, ca_w,
                        pp1_w1, pp1_b1, pp1_w2, pp1_b2, pp2_w, pp2_b)
    prompt = prompt.reshape(B, LT, N_LAYER, N_BLOCK, N_HEAD, HEAD_DIM)
    return jnp.transpose(prompt, (2, 3, 0, 4, 1, 5))
